# Initial kernel scaffold; baseline (speedup 1.0000x reference)
import jax
import jax.numpy as jnp
from jax.experimental import pallas as pl


def kernel(features, edge_index, W1, b1, W2, b2):
    raise NotImplementedError("write your pallas kernel here")


# speedup vs baseline: 4.4082x; 4.4082x over previous
import functools

import jax
import jax.numpy as jnp
from jax import lax
from jax.experimental import pallas as pl
from jax.experimental.pallas import tpu as pltpu
from jax.experimental.pallas import tpu_sc as plsc

NC = 2
NS = 16
NW = NC * NS
CH = 128


def _sc_mesh():
    return plsc.VectorSubcoreMesh(core_axis_name="c", subcore_axis_name="s")


def _deg_call(src3, dst3, npad):
    cp = src3.shape[1]
    sl = npad // NS

    @functools.partial(
        pl.kernel,
        mesh=_sc_mesh(),
        out_type=jax.ShapeDtypeStruct((NC, 2, npad), jnp.float32),
        scratch_types=[
            pltpu.VMEM_SHARED((npad,), jnp.float32),
            pltpu.VMEM_SHARED((npad,), jnp.float32),
            pltpu.VMEM((cp, CH), jnp.int32),
            pltpu.VMEM((cp, CH), jnp.int32),
            pltpu.VMEM((CH,), jnp.float32),
            pltpu.VMEM((sl,), jnp.float32),
        ],
    )
    def k(src_h, dst_h, out_h, dego, degi, sidx, didx, ones_v, zbuf):
        c = lax.axis_index("c")
        s = lax.axis_index("s")
        wid = c * NS + s
        for i in range(sl // 16):
            zbuf[pl.ds(i * 16, 16)] = jnp.zeros((16,), jnp.float32)
        for i in range(CH // 16):
            ones_v[pl.ds(i * 16, 16)] = jnp.ones((16,), jnp.float32)
        pltpu.sync_copy(zbuf, dego.at[pl.ds(s * sl, sl)])
        pltpu.sync_copy(zbuf, degi.at[pl.ds(s * sl, sl)])
        pltpu.sync_copy(src_h.at[wid], sidx)
        pltpu.sync_copy(dst_h.at[wid], didx)
        plsc.subcore_barrier()
        for j in range(cp):
            pltpu.sync_copy(ones_v, dego.at[sidx.at[j]], add=True)
            pltpu.sync_copy(ones_v, degi.at[didx.at[j]], add=True)
        plsc.subcore_barrier()
        pltpu.sync_copy(dego.at[pl.ds(s * sl, sl)], out_h.at[c, 0, pl.ds(s * sl, sl)])
        pltpu.sync_copy(degi.at[pl.ds(s * sl, sl)], out_h.at[c, 1, pl.ds(s * sl, sl)])

    return k(src3, dst3)


def _agg_call(h, src3, dst3, zeros):
    npad, d = h.shape
    cp = src3.shape[1]
    sl = npad // NS

    @functools.partial(
        pl.kernel,
        mesh=_sc_mesh(),
        out_type=jax.ShapeDtypeStruct((NC, npad, d), jnp.float32),
        scratch_types=[
            pltpu.VMEM_SHARED((npad, d), jnp.float32),
            pltpu.VMEM((cp, CH), jnp.int32),
            pltpu.VMEM((cp, CH), jnp.int32),
            pltpu.VMEM((CH, d), jnp.float32),
        ],
    )
    def k(h_h, src_h, dst_h, z_h, out_h, acc, sidx, didx, rows):
        c = lax.axis_index("c")
        s = lax.axis_index("s")
        wid = c * NS + s
        pltpu.sync_copy(z_h.at[pl.ds(s * sl, sl)], acc.at[pl.ds(s * sl, sl)])
        pltpu.sync_copy(src_h.at[wid], sidx)
        pltpu.sync_copy(dst_h.at[wid], didx)
        plsc.subcore_barrier()
        for j in range(cp):
            pltpu.sync_copy(h_h.at[sidx.at[j]], rows)
            pltpu.sync_copy(rows, acc.at[didx.at[j]], add=True)
        plsc.subcore_barrier()
        pltpu.sync_copy(acc.at[pl.ds(s * sl, sl)], out_h.at[c, pl.ds(s * sl, sl)])

    return k(h, src3, dst3, zeros)


def _tc1(feat, w1, dp4):
    npad, f = feat.shape
    h = w1.shape[1]
    br = 512
    g = npad // br

    def body(f_ref, w_ref, dp_ref, h_ref, n_ref):
        dsrc = dp_ref[0, 0] + dp_ref[1, 0]
        ddst = dp_ref[0, 1] + dp_ref[1, 1]
        ns = lax.rsqrt(jnp.maximum(dsrc, 1.0))
        nd = lax.rsqrt(jnp.maximum(ddst, 1.0))
        n_ref[0] = ns
        n_ref[1] = nd
        y = jnp.dot(f_ref[...], w_ref[...], preferred_element_type=jnp.float32)
        h_ref[...] = y * ns

    return pl.pallas_call(
        body,
        grid=(g,),
        in_specs=[
            pl.BlockSpec((br, f), lambda i: (i, 0)),
            pl.BlockSpec((f, h), lambda i: (0, 0)),
            pl.BlockSpec((NC, 2, br, 1), lambda i: (0, 0, i, 0)),
        ],
        out_specs=[
            pl.BlockSpec((br, h), lambda i: (i, 0)),
            pl.BlockSpec((2, br, 1), lambda i: (0, i, 0)),
        ],
        out_shape=[
            jax.ShapeDtypeStruct((npad, h), jnp.float32),
            jax.ShapeDtypeStruct((2, npad, 1), jnp.float32),
        ],
    )(feat, w1, dp4)


def _tc2(p1, norms, b1):
    npad, h = p1.shape[1], p1.shape[2]
    br = 512
    g = npad // br

    def body(p_ref, n_ref, b_ref, o_ref):
        agg = p_ref[0] + p_ref[1]
        x = jnp.maximum(agg * n_ref[1] + b_ref[...], 0.0)
        o_ref[...] = x * n_ref[0]

    return pl.pallas_call(
        body,
        grid=(g,),
        in_specs=[
            pl.BlockSpec((NC, br, h), lambda i: (0, i, 0)),
            pl.BlockSpec((2, br, 1), lambda i: (0, i, 0)),
            pl.BlockSpec((1, h), lambda i: (0, 0)),
        ],
        out_specs=pl.BlockSpec((br, h), lambda i: (i, 0)),
        out_shape=jax.ShapeDtypeStruct((npad, h), jnp.float32),
    )(p1, norms, b1)


def _tc3(p2, norms, w2, b2, n_out):
    h = p2.shape[2]
    o = w2.shape[1]
    br = 400
    g = n_out // br

    def body(p_ref, n_ref, w_ref, b_ref, o_ref):
        agg = p_ref[0] + p_ref[1]
        y = jnp.dot(agg, w_ref[...], preferred_element_type=jnp.float32)
        o_ref[...] = y * n_ref[1] + b_ref[...]

    return pl.pallas_call(
        body,
        grid=(g,),
        in_specs=[
            pl.BlockSpec((NC, br, h), lambda i: (0, i, 0)),
            pl.BlockSpec((2, br, 1), lambda i: (0, i, 0)),
            pl.BlockSpec((h, o), lambda i: (0, 0)),
            pl.BlockSpec((1, o), lambda i: (0, 0)),
        ],
        out_specs=pl.BlockSpec((br, o), lambda i: (i, 0)),
        out_shape=jax.ShapeDtypeStruct((n_out, o), jnp.float32),
    )(p2, norms, w2, b2)


def kernel(features, edge_index, W1, b1, W2, b2):
    n, f = features.shape
    e = edge_index.shape[1]
    npad = -(-(n + 1) // 2560) * 2560
    cp = -(-e // (NW * CH))
    epad = NW * CH * cp

    src = edge_index[0].astype(jnp.int32)
    dst = edge_index[1].astype(jnp.int32)
    pad_idx = jnp.full((epad - e,), n, dtype=jnp.int32)
    src3 = jnp.concatenate([src, pad_idx]).reshape(NW, cp, CH)
    dst3 = jnp.concatenate([dst, pad_idx]).reshape(NW, cp, CH)
    feat_pad = jnp.zeros((npad, f), jnp.float32).at[:n].set(features)

    dp = _deg_call(src3, dst3, npad)
    h1, norms = _tc1(feat_pad, W1, dp.reshape(NC, 2, npad, 1))
    z1 = jnp.zeros_like(h1)
    p1 = _agg_call(h1, src3, dst3, z1)
    h2 = _tc2(p1, norms, b1.reshape(1, -1))
    p2 = _agg_call(h2, src3, dst3, z1)
    return _tc3(p2, norms, W2, b2.reshape(1, -1), n)


# speedup vs baseline: 5.2672x; 1.1949x over previous
import functools

import jax
import jax.numpy as jnp
from jax import lax
from jax.experimental import pallas as pl
from jax.experimental.pallas import tpu as pltpu
from jax.experimental.pallas import tpu_sc as plsc

NC = 2
NS = 16
NW = NC * NS
CH = 128


def _sc_mesh():
    return plsc.VectorSubcoreMesh(core_axis_name="c", subcore_axis_name="s")


def _deg_call(src3, dst3, npad):
    cp = src3.shape[1]
    sl = npad // NS

    @functools.partial(
        pl.kernel,
        mesh=_sc_mesh(),
        out_type=jax.ShapeDtypeStruct((NC, 2, npad), jnp.float32),
        scratch_types=[
            pltpu.VMEM_SHARED((npad,), jnp.float32),
            pltpu.VMEM_SHARED((npad,), jnp.float32),
            pltpu.VMEM((cp, CH), jnp.int32),
            pltpu.VMEM((cp, CH), jnp.int32),
            pltpu.VMEM((CH,), jnp.float32),
            pltpu.VMEM((sl,), jnp.float32),
            pltpu.SemaphoreType.DMA,
        ],
    )
    def k(src_h, dst_h, out_h, dego, degi, sidx, didx, ones_v, zbuf, sem):
        c = lax.axis_index("c")
        s = lax.axis_index("s")
        wid = c * NS + s
        for i in range(sl // 16):
            zbuf[pl.ds(i * 16, 16)] = jnp.zeros((16,), jnp.float32)
        for i in range(CH // 16):
            ones_v[pl.ds(i * 16, 16)] = jnp.ones((16,), jnp.float32)
        pltpu.sync_copy(zbuf, dego.at[pl.ds(s * sl, sl)])
        pltpu.sync_copy(zbuf, degi.at[pl.ds(s * sl, sl)])
        pltpu.sync_copy(src_h.at[wid], sidx)
        pltpu.sync_copy(dst_h.at[wid], didx)
        plsc.subcore_barrier()
        ds_ = []
        for j in range(cp):
            ds_.append(pltpu.async_copy(ones_v, dego.at[sidx.at[j]], sem, add=True))
            ds_.append(pltpu.async_copy(ones_v, degi.at[didx.at[j]], sem, add=True))
        for d_ in ds_:
            d_.wait()
        plsc.subcore_barrier()
        pltpu.sync_copy(dego.at[pl.ds(s * sl, sl)], out_h.at[c, 0, pl.ds(s * sl, sl)])
        pltpu.sync_copy(degi.at[pl.ds(s * sl, sl)], out_h.at[c, 1, pl.ds(s * sl, sl)])

    return k(src3, dst3)


def _agg_call(h, src3, dst3, zeros):
    npad, d = h.shape
    cp = src3.shape[1]
    sl = npad // NS
    nbuf = 4

    @functools.partial(
        pl.kernel,
        mesh=_sc_mesh(),
        out_type=jax.ShapeDtypeStruct((NC, npad, d), jnp.float32),
        scratch_types=[
            pltpu.VMEM_SHARED((npad, d), jnp.float32),
            pltpu.VMEM((4, CH), jnp.int32),
            pltpu.VMEM((4, CH), jnp.int32),
            pltpu.VMEM((2, CH, d), jnp.float32),
            pltpu.SemaphoreType.DMA,
            [pltpu.SemaphoreType.DMA] * 4,
            [pltpu.SemaphoreType.DMA] * 2,
            [pltpu.SemaphoreType.DMA] * 2,
        ],
    )
    def k(h_h, src_h, dst_h, z_h, out_h, acc, sidx, didx, rows, zsem, isems, gsems, ssems):
        c = lax.axis_index("c")
        s = lax.axis_index("s")
        wid = c * NS + s
        zd = pltpu.async_copy(z_h.at[pl.ds(s * sl, sl)], acc.at[pl.ds(s * sl, sl)], zsem)

        idd, gd, sd = {}, {}, {}

        def load_idx(j):
            r = j % 4
            idd[j] = (
                pltpu.async_copy(src_h.at[wid, pl.ds(j, 1)], sidx.at[pl.ds(r, 1)], isems[r]),
                pltpu.async_copy(dst_h.at[wid, pl.ds(j, 1)], didx.at[pl.ds(r, 1)], isems[r]),
            )

        def gather(j):
            gd[j] = pltpu.async_copy(h_h.at[sidx.at[j % 4]], rows.at[j % 2], gsems[j % 2])

        for t in range(min(3, cp)):
            load_idx(t)
        idd[0][0].wait()
        idd[0][1].wait()
        gather(0)
        zd.wait()
        plsc.subcore_barrier()
        for j in range(cp):
            if j >= 1:
                sd[j - 1].wait()
            if j + 3 < cp:
                load_idx(j + 3)
            if j + 1 < cp:
                idd[j + 1][0].wait()
                idd[j + 1][1].wait()
                gather(j + 1)
            gd[j].wait()
            sd[j] = pltpu.async_copy(
                rows.at[j % 2], acc.at[didx.at[j % 4]], ssems[j % 2], add=True)
        sd[cp - 1].wait()
        plsc.subcore_barrier()
        pltpu.sync_copy(acc.at[pl.ds(s * sl, sl)], out_h.at[c, pl.ds(s * sl, sl)])

    return k(h, src3, dst3, zeros)


def _tc1(feat, w1, dp4):
    npad, f = feat.shape
    h = w1.shape[1]
    br = 512
    g = npad // br

    def body(f_ref, w_ref, dp_ref, h_ref, n_ref):
        dsrc = dp_ref[0, 0] + dp_ref[1, 0]
        ddst = dp_ref[0, 1] + dp_ref[1, 1]
        ns = lax.rsqrt(jnp.maximum(dsrc, 1.0))
        nd = lax.rsqrt(jnp.maximum(ddst, 1.0))
        n_ref[0] = ns
        n_ref[1] = nd
        y = jnp.dot(f_ref[...], w_ref[...], preferred_element_type=jnp.float32)
        h_ref[...] = y * ns

    return pl.pallas_call(
        body,
        grid=(g,),
        in_specs=[
            pl.BlockSpec((br, f), lambda i: (i, 0)),
            pl.BlockSpec((f, h), lambda i: (0, 0)),
            pl.BlockSpec((NC, 2, br, 1), lambda i: (0, 0, i, 0)),
        ],
        out_specs=[
            pl.BlockSpec((br, h), lambda i: (i, 0)),
            pl.BlockSpec((2, br, 1), lambda i: (0, i, 0)),
        ],
        out_shape=[
            jax.ShapeDtypeStruct((npad, h), jnp.float32),
            jax.ShapeDtypeStruct((2, npad, 1), jnp.float32),
        ],
    )(feat, w1, dp4)


def _tc2(p1, norms, b1):
    npad, h = p1.shape[1], p1.shape[2]
    br = 512
    g = npad // br

    def body(p_ref, n_ref, b_ref, o_ref):
        agg = p_ref[0] + p_ref[1]
        x = jnp.maximum(agg * n_ref[1] + b_ref[...], 0.0)
        o_ref[...] = x * n_ref[0]

    return pl.pallas_call(
        body,
        grid=(g,),
        in_specs=[
            pl.BlockSpec((NC, br, h), lambda i: (0, i, 0)),
            pl.BlockSpec((2, br, 1), lambda i: (0, i, 0)),
            pl.BlockSpec((1, h), lambda i: (0, 0)),
        ],
        out_specs=pl.BlockSpec((br, h), lambda i: (i, 0)),
        out_shape=jax.ShapeDtypeStruct((npad, h), jnp.float32),
    )(p1, norms, b1)


def _tc3(p2, norms, w2, b2, n_out):
    h = p2.shape[2]
    o = w2.shape[1]
    br = 400
    g = n_out // br

    def body(p_ref, n_ref, w_ref, b_ref, o_ref):
        agg = p_ref[0] + p_ref[1]
        y = jnp.dot(agg, w_ref[...], preferred_element_type=jnp.float32)
        o_ref[...] = y * n_ref[1] + b_ref[...]

    return pl.pallas_call(
        body,
        grid=(g,),
        in_specs=[
            pl.BlockSpec((NC, br, h), lambda i: (0, i, 0)),
            pl.BlockSpec((2, br, 1), lambda i: (0, i, 0)),
            pl.BlockSpec((h, o), lambda i: (0, 0)),
            pl.BlockSpec((1, o), lambda i: (0, 0)),
        ],
        out_specs=pl.BlockSpec((br, o), lambda i: (i, 0)),
        out_shape=jax.ShapeDtypeStruct((n_out, o), jnp.float32),
    )(p2, norms, w2, b2)


def kernel(features, edge_index, W1, b1, W2, b2):
    n, f = features.shape
    e = edge_index.shape[1]
    npad = -(-(n + 1) // 2560) * 2560
    cp = -(-e // (NW * CH))
    epad = NW * CH * cp

    src = edge_index[0].astype(jnp.int32)
    dst = edge_index[1].astype(jnp.int32)
    pad_idx = jnp.full((epad - e,), n, dtype=jnp.int32)
    src3 = jnp.concatenate([src, pad_idx]).reshape(NW, cp, CH)
    dst3 = jnp.concatenate([dst, pad_idx]).reshape(NW, cp, CH)
    feat_pad = jnp.zeros((npad, f), jnp.float32).at[:n].set(features)

    dp = _deg_call(src3, dst3, npad)
    h1, norms = _tc1(feat_pad, W1, dp.reshape(NC, 2, npad, 1))
    z1 = jnp.zeros_like(h1)
    p1 = _agg_call(h1, src3, dst3, z1)
    h2 = _tc2(p1, norms, b1.reshape(1, -1))
    p2 = _agg_call(h2, src3, dst3, z1)
    return _tc3(p2, norms, W2, b2.reshape(1, -1), n)


# speedup vs baseline: 10.9592x; 2.0807x over previous
import functools

import jax
import jax.numpy as jnp
from jax import lax
from jax.experimental import pallas as pl
from jax.experimental.pallas import tpu as pltpu
from jax.experimental.pallas import tpu_sc as plsc

NC = 2
NS = 16
NW = NC * NS
CH = 128


def _sc_mesh():
    return plsc.VectorSubcoreMesh(core_axis_name="c", subcore_axis_name="s")


def _deg_call(src3, dst3, npad):
    cp = src3.shape[1]
    sl = npad // NS

    @functools.partial(
        pl.kernel,
        mesh=_sc_mesh(),
        out_type=jax.ShapeDtypeStruct((NC, 2, npad), jnp.float32),
        scratch_types=[
            pltpu.VMEM_SHARED((npad,), jnp.float32),
            pltpu.VMEM_SHARED((npad,), jnp.float32),
            pltpu.VMEM((cp, CH), jnp.int32),
            pltpu.VMEM((cp, CH), jnp.int32),
            pltpu.VMEM((CH,), jnp.float32),
            pltpu.VMEM((sl,), jnp.float32),
            pltpu.SemaphoreType.DMA,
        ],
    )
    def k(src_h, dst_h, out_h, dego, degi, sidx, didx, ones_v, zbuf, sem):
        c = lax.axis_index("c")
        s = lax.axis_index("s")
        wid = c * NS + s
        for i in range(sl // 16):
            zbuf[pl.ds(i * 16, 16)] = jnp.zeros((16,), jnp.float32)
        for i in range(CH // 16):
            ones_v[pl.ds(i * 16, 16)] = jnp.ones((16,), jnp.float32)
        pltpu.sync_copy(zbuf, dego.at[pl.ds(s * sl, sl)])
        pltpu.sync_copy(zbuf, degi.at[pl.ds(s * sl, sl)])
        pltpu.sync_copy(src_h.at[wid], sidx)
        pltpu.sync_copy(dst_h.at[wid], didx)
        plsc.subcore_barrier()
        ds_ = []
        for j in range(cp):
            ds_.append(pltpu.async_copy(ones_v, dego.at[sidx.at[j]], sem, add=True))
            ds_.append(pltpu.async_copy(ones_v, degi.at[didx.at[j]], sem, add=True))
        for d_ in ds_:
            d_.wait()
        plsc.subcore_barrier()
        pltpu.sync_copy(dego.at[pl.ds(s * sl, sl)], out_h.at[c, 0, pl.ds(s * sl, sl)])
        pltpu.sync_copy(degi.at[pl.ds(s * sl, sl)], out_h.at[c, 1, pl.ds(s * sl, sl)])

    return k(src3, dst3)


def _agg_call(h, src3, dst3, zeros):
    npad, d = h.shape
    cp = src3.shape[1]
    sl = npad // NS
    nbuf = 4

    @functools.partial(
        pl.kernel,
        mesh=_sc_mesh(),
        out_type=jax.ShapeDtypeStruct((NC, npad, d), jnp.float32),
        scratch_types=[
            pltpu.VMEM_SHARED((npad, d), jnp.float32),
            pltpu.VMEM((4, CH), jnp.int32),
            pltpu.VMEM((4, CH), jnp.int32),
            pltpu.VMEM((2, CH, d), jnp.float32),
            pltpu.SemaphoreType.DMA,
            [pltpu.SemaphoreType.DMA] * 4,
            [pltpu.SemaphoreType.DMA] * 2,
            [pltpu.SemaphoreType.DMA] * 2,
        ],
    )
    def k(h_h, src_h, dst_h, z_h, out_h, acc, sidx, didx, rows, zsem, isems, gsems, ssems):
        c = lax.axis_index("c")
        s = lax.axis_index("s")
        wid = c * NS + s
        zd = pltpu.async_copy(z_h.at[pl.ds(s * sl, sl)], acc.at[pl.ds(s * sl, sl)], zsem)

        idd, gd, sd = {}, {}, {}

        def load_idx(j):
            r = j % 4
            idd[j] = (
                pltpu.async_copy(src_h.at[wid, pl.ds(j, 1)], sidx.at[pl.ds(r, 1)], isems[r]),
                pltpu.async_copy(dst_h.at[wid, pl.ds(j, 1)], didx.at[pl.ds(r, 1)], isems[r]),
            )

        def gather(j):
            gd[j] = pltpu.async_copy(h_h.at[sidx.at[j % 4]], rows.at[j % 2], gsems[j % 2])

        for t in range(min(3, cp)):
            load_idx(t)
        idd[0][0].wait()
        idd[0][1].wait()
        gather(0)
        zd.wait()
        plsc.subcore_barrier()
        for j in range(cp):
            if j >= 1:
                sd[j - 1].wait()
            if j + 3 < cp:
                load_idx(j + 3)
            if j + 1 < cp:
                idd[j + 1][0].wait()
                idd[j + 1][1].wait()
                gather(j + 1)
            gd[j].wait()
            sd[j] = pltpu.async_copy(
                rows.at[j % 2], acc.at[didx.at[j % 4]], ssems[j % 2], add=True)
        sd[cp - 1].wait()
        plsc.subcore_barrier()
        pltpu.sync_copy(acc.at[pl.ds(s * sl, sl)], out_h.at[c, pl.ds(s * sl, sl)])

    return k(h, src3, dst3, zeros)


def _tc1(feat, w1, dp4):
    npad, f = feat.shape
    h = w1.shape[1]
    br = 512
    g = npad // br

    def body(f_ref, w_ref, dp_ref, h_ref, n_ref):
        dsrc = dp_ref[0, 0] + dp_ref[1, 0]
        ddst = dp_ref[0, 1] + dp_ref[1, 1]
        ns = lax.rsqrt(jnp.maximum(dsrc, 1.0))
        nd = lax.rsqrt(jnp.maximum(ddst, 1.0))
        n_ref[0] = ns
        n_ref[1] = nd
        y = jnp.dot(f_ref[...], w_ref[...], preferred_element_type=jnp.float32)
        h_ref[...] = y * ns

    return pl.pallas_call(
        body,
        grid=(g,),
        in_specs=[
            pl.BlockSpec((br, f), lambda i: (i, 0)),
            pl.BlockSpec((f, h), lambda i: (0, 0)),
            pl.BlockSpec((NC, 2, br, 1), lambda i: (0, 0, i, 0)),
        ],
        out_specs=[
            pl.BlockSpec((br, h), lambda i: (i, 0)),
            pl.BlockSpec((2, br, 1), lambda i: (0, i, 0)),
        ],
        out_shape=[
            jax.ShapeDtypeStruct((npad, h), jnp.float32),
            jax.ShapeDtypeStruct((2, npad, 1), jnp.float32),
        ],
    )(feat, w1, dp4)


def _tc2(p1, norms, b1):
    npad, h = p1.shape[1], p1.shape[2]
    br = 512
    g = npad // br

    def body(p_ref, n_ref, b_ref, o_ref):
        agg = p_ref[0] + p_ref[1]
        x = jnp.maximum(agg * n_ref[1] + b_ref[...], 0.0)
        o_ref[...] = x * n_ref[0]

    return pl.pallas_call(
        body,
        grid=(g,),
        in_specs=[
            pl.BlockSpec((NC, br, h), lambda i: (0, i, 0)),
            pl.BlockSpec((2, br, 1), lambda i: (0, i, 0)),
            pl.BlockSpec((1, h), lambda i: (0, 0)),
        ],
        out_specs=pl.BlockSpec((br, h), lambda i: (i, 0)),
        out_shape=jax.ShapeDtypeStruct((npad, h), jnp.float32),
    )(p1, norms, b1)


def _tc3(p2, norms, w2, b2, n_out):
    h = p2.shape[2]
    o = w2.shape[1]
    br = 400
    g = n_out // br

    def body(p_ref, n_ref, w_ref, b_ref, o_ref):
        agg = p_ref[0] + p_ref[1]
        y = jnp.dot(agg, w_ref[...], preferred_element_type=jnp.float32)
        o_ref[...] = y * n_ref[1] + b_ref[...]

    return pl.pallas_call(
        body,
        grid=(g,),
        in_specs=[
            pl.BlockSpec((NC, br, h), lambda i: (0, i, 0)),
            pl.BlockSpec((2, br, 1), lambda i: (0, i, 0)),
            pl.BlockSpec((h, o), lambda i: (0, 0)),
            pl.BlockSpec((1, o), lambda i: (0, 0)),
        ],
        out_specs=pl.BlockSpec((br, o), lambda i: (i, 0)),
        out_shape=jax.ShapeDtypeStruct((n_out, o), jnp.float32),
    )(p2, norms, w2, b2)


def kernel(features, edge_index, W1, b1, W2, b2):
    n, f = features.shape
    e = edge_index.shape[1]
    npad = -(-(n + 1) // 2560) * 2560
    cp = -(-e // (NW * CH))
    epad = NW * CH * cp

    src = edge_index[0].astype(jnp.int32)
    dst = edge_index[1].astype(jnp.int32)
    pad_idx = n + (jnp.arange(epad - e, dtype=jnp.int32) % (npad - n))
    src3 = jnp.concatenate([src, pad_idx]).reshape(NW, cp, CH)
    dst3 = jnp.concatenate([dst, pad_idx]).reshape(NW, cp, CH)
    feat_pad = jnp.zeros((npad, f), jnp.float32).at[:n].set(features)

    dp = _deg_call(src3, dst3, npad)
    h1, norms = _tc1(feat_pad, W1, dp.reshape(NC, 2, npad, 1))
    z1 = jnp.zeros_like(h1)
    p1 = _agg_call(h1, src3, dst3, z1)
    h2 = _tc2(p1, norms, b1.reshape(1, -1))
    p2 = _agg_call(h2, src3, dst3, z1)
    return _tc3(p2, norms, W2, b2.reshape(1, -1), n)


# speedup vs baseline: 11.8009x; 1.0768x over previous
import functools

import jax
import jax.numpy as jnp
from jax import lax
from jax.experimental import pallas as pl
from jax.experimental.pallas import tpu as pltpu
from jax.experimental.pallas import tpu_sc as plsc

NC = 2
NS = 16
NW = NC * NS
CH = 128


def _sc_mesh():
    return plsc.VectorSubcoreMesh(core_axis_name="c", subcore_axis_name="s")


def _deg_call(src3, dst3, npad):
    cp = src3.shape[1]
    sl = npad // NS

    @functools.partial(
        pl.kernel,
        mesh=_sc_mesh(),
        out_type=jax.ShapeDtypeStruct((NC, 2, npad), jnp.float32),
        scratch_types=[
            pltpu.VMEM_SHARED((npad,), jnp.float32),
            pltpu.VMEM_SHARED((npad,), jnp.float32),
            pltpu.VMEM((cp, CH), jnp.int32),
            pltpu.VMEM((cp, CH), jnp.int32),
            pltpu.VMEM((CH,), jnp.float32),
            pltpu.VMEM((sl,), jnp.float32),
            pltpu.SemaphoreType.DMA,
        ],
    )
    def k(src_h, dst_h, out_h, dego, degi, sidx, didx, ones_v, zbuf, sem):
        c = lax.axis_index("c")
        s = lax.axis_index("s")
        wid = c * NS + s
        for i in range(sl // 16):
            zbuf[pl.ds(i * 16, 16)] = jnp.zeros((16,), jnp.float32)
        for i in range(CH // 16):
            ones_v[pl.ds(i * 16, 16)] = jnp.ones((16,), jnp.float32)
        pltpu.sync_copy(zbuf, dego.at[pl.ds(s * sl, sl)])
        pltpu.sync_copy(zbuf, degi.at[pl.ds(s * sl, sl)])
        pltpu.sync_copy(src_h.at[wid], sidx)
        pltpu.sync_copy(dst_h.at[wid], didx)
        plsc.subcore_barrier()
        ds_ = []
        for j in range(cp):
            ds_.append(pltpu.async_copy(ones_v, dego.at[sidx.at[j]], sem, add=True))
            ds_.append(pltpu.async_copy(ones_v, degi.at[didx.at[j]], sem, add=True))
        for d_ in ds_:
            d_.wait()
        plsc.subcore_barrier()
        pltpu.sync_copy(dego.at[pl.ds(s * sl, sl)], out_h.at[c, 0, pl.ds(s * sl, sl)])
        pltpu.sync_copy(degi.at[pl.ds(s * sl, sl)], out_h.at[c, 1, pl.ds(s * sl, sl)])

    return k(src3, dst3)


def _agg_call(h, src3, dst3, zeros):
    npad, d = h.shape
    cp = src3.shape[1]
    sl = npad // NS
    nbuf = 4

    @functools.partial(
        pl.kernel,
        mesh=_sc_mesh(),
        compiler_params=pltpu.CompilerParams(use_tc_tiling_on_sc=False),
        out_type=jax.ShapeDtypeStruct((NC, npad, d), jnp.float32),
        scratch_types=[
            pltpu.VMEM_SHARED((npad, d), jnp.float32),
            pltpu.VMEM((4, CH), jnp.int32),
            pltpu.VMEM((4, CH), jnp.int32),
            pltpu.VMEM((2, CH, d), jnp.float32),
            pltpu.SemaphoreType.DMA,
            [pltpu.SemaphoreType.DMA] * 4,
            [pltpu.SemaphoreType.DMA] * 2,
            [pltpu.SemaphoreType.DMA] * 2,
        ],
    )
    def k(h_h, src_h, dst_h, z_h, out_h, acc, sidx, didx, rows, zsem, isems, gsems, ssems):
        c = lax.axis_index("c")
        s = lax.axis_index("s")
        wid = c * NS + s
        zd = pltpu.async_copy(z_h.at[pl.ds(s * sl, sl)], acc.at[pl.ds(s * sl, sl)], zsem)

        idd, gd, sd = {}, {}, {}

        def load_idx(j):
            r = j % 4
            idd[j] = (
                pltpu.async_copy(src_h.at[wid, pl.ds(j, 1)], sidx.at[pl.ds(r, 1)], isems[r]),
                pltpu.async_copy(dst_h.at[wid, pl.ds(j, 1)], didx.at[pl.ds(r, 1)], isems[r]),
            )

        def gather(j):
            gd[j] = pltpu.async_copy(h_h.at[sidx.at[j % 4]], rows.at[j % 2], gsems[j % 2])

        for t in range(min(3, cp)):
            load_idx(t)
        idd[0][0].wait()
        idd[0][1].wait()
        gather(0)
        zd.wait()
        plsc.subcore_barrier()
        for j in range(cp):
            if j >= 1:
                sd[j - 1].wait()
            if j + 3 < cp:
                load_idx(j + 3)
            if j + 1 < cp:
                idd[j + 1][0].wait()
                idd[j + 1][1].wait()
                gather(j + 1)
            gd[j].wait()
            sd[j] = pltpu.async_copy(
                rows.at[j % 2], acc.at[didx.at[j % 4]], ssems[j % 2], add=True)
        sd[cp - 1].wait()
        plsc.subcore_barrier()
        pltpu.sync_copy(acc.at[pl.ds(s * sl, sl)], out_h.at[c, pl.ds(s * sl, sl)])

    return k(h, src3, dst3, zeros)


def _tc1(feat, w1, dp4):
    npad, f = feat.shape
    h = w1.shape[1]
    br = 512
    g = npad // br

    def body(f_ref, w_ref, dp_ref, h_ref, n_ref):
        dsrc = dp_ref[0, 0] + dp_ref[1, 0]
        ddst = dp_ref[0, 1] + dp_ref[1, 1]
        ns = lax.rsqrt(jnp.maximum(dsrc, 1.0))
        nd = lax.rsqrt(jnp.maximum(ddst, 1.0))
        n_ref[0] = ns
        n_ref[1] = nd
        y = jnp.dot(f_ref[...], w_ref[...], preferred_element_type=jnp.float32)
        h_ref[...] = y * ns

    return pl.pallas_call(
        body,
        grid=(g,),
        in_specs=[
            pl.BlockSpec((br, f), lambda i: (i, 0)),
            pl.BlockSpec((f, h), lambda i: (0, 0)),
            pl.BlockSpec((NC, 2, br, 1), lambda i: (0, 0, i, 0)),
        ],
        out_specs=[
            pl.BlockSpec((br, h), lambda i: (i, 0)),
            pl.BlockSpec((2, br, 1), lambda i: (0, i, 0)),
        ],
        out_shape=[
            jax.ShapeDtypeStruct((npad, h), jnp.float32),
            jax.ShapeDtypeStruct((2, npad, 1), jnp.float32),
        ],
    )(feat, w1, dp4)


def _tc2(p1, norms, b1, w2):
    npad, h = p1.shape[1], p1.shape[2]
    o = w2.shape[1]
    br = 512
    g = npad // br

    def body(p_ref, n_ref, b_ref, w_ref, o_ref):
        agg = p_ref[0] + p_ref[1]
        x = jnp.maximum(agg * n_ref[1] + b_ref[...], 0.0)
        y = jnp.dot(x, w_ref[...], preferred_element_type=jnp.float32)
        o_ref[...] = y * n_ref[0]

    return pl.pallas_call(
        body,
        grid=(g,),
        in_specs=[
            pl.BlockSpec((NC, br, h), lambda i: (0, i, 0)),
            pl.BlockSpec((2, br, 1), lambda i: (0, i, 0)),
            pl.BlockSpec((1, h), lambda i: (0, 0)),
            pl.BlockSpec((h, o), lambda i: (0, 0)),
        ],
        out_specs=pl.BlockSpec((br, o), lambda i: (i, 0)),
        out_shape=jax.ShapeDtypeStruct((npad, o), jnp.float32),
    )(p1, norms, b1, w2)


def _tc3(p2, norms, b2, n_out):
    o = p2.shape[2]
    br = 400
    g = n_out // br

    def body(p_ref, n_ref, b_ref, o_ref):
        o_ref[...] = (p_ref[0] + p_ref[1]) * n_ref[1] + b_ref[...]

    return pl.pallas_call(
        body,
        grid=(g,),
        in_specs=[
            pl.BlockSpec((NC, br, o), lambda i: (0, i, 0)),
            pl.BlockSpec((2, br, 1), lambda i: (0, i, 0)),
            pl.BlockSpec((1, o), lambda i: (0, 0)),
        ],
        out_specs=pl.BlockSpec((br, o), lambda i: (i, 0)),
        out_shape=jax.ShapeDtypeStruct((n_out, o), jnp.float32),
    )(p2, norms, b2)


def kernel(features, edge_index, W1, b1, W2, b2):
    n, f = features.shape
    e = edge_index.shape[1]
    npad = -(-(n + 1) // 2560) * 2560
    cp = -(-e // (NW * CH))
    epad = NW * CH * cp

    src = edge_index[0].astype(jnp.int32)
    dst = edge_index[1].astype(jnp.int32)
    pad_idx = n + (jnp.arange(epad - e, dtype=jnp.int32) % (npad - n))
    src3 = jnp.concatenate([src, pad_idx]).reshape(NW, cp, CH)
    dst3 = jnp.concatenate([dst, pad_idx]).reshape(NW, cp, CH)
    feat_pad = jnp.zeros((npad, f), jnp.float32).at[:n].set(features)

    dp = _deg_call(src3, dst3, npad)
    h1, norms = _tc1(feat_pad, W1, dp.reshape(NC, 2, npad, 1))
    z1 = jnp.zeros_like(h1)
    p1 = _agg_call(h1, src3, dst3, z1)
    h2 = _tc2(p1, norms, b1.reshape(1, -1), W2)
    z2 = jnp.zeros_like(h2)
    p2 = _agg_call(h2, src3, dst3, z2)
    return _tc3(p2, norms, b2.reshape(1, -1), n)


# speedup vs baseline: 12.6954x; 1.0758x over previous
import functools

import jax
import jax.numpy as jnp
from jax import lax
from jax.experimental import pallas as pl
from jax.experimental.pallas import tpu as pltpu
from jax.experimental.pallas import tpu_sc as plsc

NC = 2
NS = 16
NW = NC * NS
CH = 128


def _sc_mesh():
    return plsc.VectorSubcoreMesh(core_axis_name="c", subcore_axis_name="s")


def _deg_call(src3, dst3, npad):
    cp = src3.shape[1]
    sl = npad // NS

    @functools.partial(
        pl.kernel,
        mesh=_sc_mesh(),
        out_type=jax.ShapeDtypeStruct((NC, 2, npad), jnp.float32),
        scratch_types=[
            pltpu.VMEM_SHARED((npad,), jnp.float32),
            pltpu.VMEM_SHARED((npad,), jnp.float32),
            pltpu.VMEM((cp, CH), jnp.int32),
            pltpu.VMEM((cp, CH), jnp.int32),
            pltpu.VMEM((CH,), jnp.float32),
            pltpu.VMEM((sl,), jnp.float32),
            pltpu.SemaphoreType.DMA,
        ],
    )
    def k(src_h, dst_h, out_h, dego, degi, sidx, didx, ones_v, zbuf, sem):
        c = lax.axis_index("c")
        s = lax.axis_index("s")
        wid = c * NS + s
        for i in range(sl // 16):
            zbuf[pl.ds(i * 16, 16)] = jnp.zeros((16,), jnp.float32)
        for i in range(CH // 16):
            ones_v[pl.ds(i * 16, 16)] = jnp.ones((16,), jnp.float32)
        pltpu.sync_copy(zbuf, dego.at[pl.ds(s * sl, sl)])
        pltpu.sync_copy(zbuf, degi.at[pl.ds(s * sl, sl)])
        pltpu.sync_copy(src_h.at[wid], sidx)
        pltpu.sync_copy(dst_h.at[wid], didx)
        plsc.subcore_barrier()
        ds_ = []
        for j in range(cp):
            ds_.append(pltpu.async_copy(ones_v, dego.at[sidx.at[j]], sem, add=True))
            ds_.append(pltpu.async_copy(ones_v, degi.at[didx.at[j]], sem, add=True))
        for d_ in ds_:
            d_.wait()
        plsc.subcore_barrier()
        pltpu.sync_copy(dego.at[pl.ds(s * sl, sl)], out_h.at[c, 0, pl.ds(s * sl, sl)])
        pltpu.sync_copy(degi.at[pl.ds(s * sl, sl)], out_h.at[c, 1, pl.ds(s * sl, sl)])

    return k(src3, dst3)


def _agg_call(h, src3, dst3, zeros):
    npad, d = h.shape
    cp = src3.shape[1]
    sl = npad // NS
    nbuf = 4

    @functools.partial(
        pl.kernel,
        mesh=_sc_mesh(),
        compiler_params=pltpu.CompilerParams(use_tc_tiling_on_sc=False),
        out_type=jax.ShapeDtypeStruct((NC, npad, d), jnp.float32),
        scratch_types=[
            pltpu.VMEM_SHARED((npad, d), jnp.float32),
            pltpu.VMEM((4, CH), jnp.int32),
            pltpu.VMEM((4, CH), jnp.int32),
            pltpu.VMEM((2, CH, d), jnp.float32),
            pltpu.SemaphoreType.DMA,
            [pltpu.SemaphoreType.DMA] * 4,
            [pltpu.SemaphoreType.DMA] * 2,
            [pltpu.SemaphoreType.DMA] * 2,
        ],
    )
    def k(h_h, src_h, dst_h, z_h, out_h, acc, sidx, didx, rows, zsem, isems, gsems, ssems):
        c = lax.axis_index("c")
        s = lax.axis_index("s")
        wid = c * NS + s
        zd = pltpu.async_copy(z_h.at[pl.ds(s * sl, sl)], acc.at[pl.ds(s * sl, sl)], zsem)

        idd, gd, sd = {}, {}, {}

        def load_idx(j):
            r = j % 4
            idd[j] = (
                pltpu.async_copy(src_h.at[wid, pl.ds(j, 1)], sidx.at[pl.ds(r, 1)], isems[r]),
                pltpu.async_copy(dst_h.at[wid, pl.ds(j, 1)], didx.at[pl.ds(r, 1)], isems[r]),
            )

        def gather(j):
            gd[j] = pltpu.async_copy(h_h.at[sidx.at[j % 4]], rows.at[j % 2], gsems[j % 2])

        for t in range(min(3, cp)):
            load_idx(t)
        idd[0][0].wait()
        idd[0][1].wait()
        gather(0)
        zd.wait()
        plsc.subcore_barrier()
        for j in range(cp):
            if j >= 1:
                sd[j - 1].wait()
            if j + 3 < cp:
                load_idx(j + 3)
            if j + 1 < cp:
                idd[j + 1][0].wait()
                idd[j + 1][1].wait()
                gather(j + 1)
            gd[j].wait()
            sd[j] = pltpu.async_copy(
                rows.at[j % 2], acc.at[didx.at[j % 4]], ssems[j % 2], add=True)
        sd[cp - 1].wait()
        plsc.subcore_barrier()
        pltpu.sync_copy(acc.at[pl.ds(s * sl, sl)], out_h.at[c, pl.ds(s * sl, sl)])

    return k(h, src3, dst3, zeros)


def _row_scale(y, ns):
    r, c = y.shape
    eye = (jax.lax.broadcasted_iota(jnp.int32, (128, 128), 0)
           == jax.lax.broadcasted_iota(jnp.int32, (128, 128), 1)).astype(jnp.float32)
    parts = []
    for k in range(r // 128):
        dg = eye * ns[k * 128:(k + 1) * 128][None, :]
        parts.append(jnp.dot(dg, y[k * 128:(k + 1) * 128, :],
                             preferred_element_type=jnp.float32))
    return jnp.concatenate(parts, axis=0)


def _tc1(feat, w1, dp):
    npad, f = feat.shape
    h = w1.shape[1]
    br = 512
    g = npad // br

    def body(f_ref, w_ref, dp_ref, h_ref, n_ref):
        dsrc = dp_ref[0, 0] + dp_ref[1, 0]
        ddst = dp_ref[0, 1] + dp_ref[1, 1]
        ns = lax.rsqrt(jnp.maximum(dsrc, 1.0))
        nd = lax.rsqrt(jnp.maximum(ddst, 1.0))
        n_ref[0, :] = ns
        n_ref[1, :] = nd
        y = jnp.dot(f_ref[...], w_ref[...], preferred_element_type=jnp.float32)
        h_ref[...] = _row_scale(y, ns)

    return pl.pallas_call(
        body,
        grid=(g,),
        in_specs=[
            pl.BlockSpec((br, f), lambda i: (i, 0)),
            pl.BlockSpec((f, h), lambda i: (0, 0)),
            pl.BlockSpec((NC, 2, br), lambda i: (0, 0, i)),
        ],
        out_specs=[
            pl.BlockSpec((br, h), lambda i: (i, 0)),
            pl.BlockSpec((2, br), lambda i: (0, i)),
        ],
        out_shape=[
            jax.ShapeDtypeStruct((npad, h), jnp.float32),
            jax.ShapeDtypeStruct((2, npad), jnp.float32),
        ],
    )(feat, w1, dp)


def _tc2(p1, norms, b1, w2):
    npad, h = p1.shape[1], p1.shape[2]
    o = w2.shape[1]
    br = 512
    g = npad // br

    def body(p_ref, n_ref, b_ref, w_ref, o_ref):
        agg = p_ref[0] + p_ref[1]
        x = jnp.maximum(_row_scale(agg, n_ref[1, :]) + b_ref[...], 0.0)
        y = jnp.dot(x, w_ref[...], preferred_element_type=jnp.float32)
        o_ref[...] = _row_scale(y, n_ref[0, :])

    return pl.pallas_call(
        body,
        grid=(g,),
        in_specs=[
            pl.BlockSpec((NC, br, h), lambda i: (0, i, 0)),
            pl.BlockSpec((2, br), lambda i: (0, i)),
            pl.BlockSpec((1, h), lambda i: (0, 0)),
            pl.BlockSpec((h, o), lambda i: (0, 0)),
        ],
        out_specs=pl.BlockSpec((br, o), lambda i: (i, 0)),
        out_shape=jax.ShapeDtypeStruct((npad, o), jnp.float32),
    )(p1, norms, b1, w2)


def _tc3(p2, norms, b2, n_out):
    npad, o = p2.shape[1], p2.shape[2]
    br = 512
    g = npad // br

    def body(p_ref, n_ref, b_ref, o_ref):
        o_ref[...] = _row_scale(p_ref[0] + p_ref[1], n_ref[1, :]) + b_ref[...]

    return pl.pallas_call(
        body,
        grid=(g,),
        in_specs=[
            pl.BlockSpec((NC, br, o), lambda i: (0, i, 0)),
            pl.BlockSpec((2, br), lambda i: (0, i)),
            pl.BlockSpec((1, o), lambda i: (0, 0)),
        ],
        out_specs=pl.BlockSpec((br, o), lambda i: (i, 0)),
        out_shape=jax.ShapeDtypeStruct((n_out, o), jnp.float32),
    )(p2, norms, b2)


def kernel(features, edge_index, W1, b1, W2, b2):
    n, f = features.shape
    e = edge_index.shape[1]
    npad = -(-(n + 1) // 2560) * 2560
    cp = -(-e // (NW * CH))
    epad = NW * CH * cp

    src = edge_index[0].astype(jnp.int32)
    dst = edge_index[1].astype(jnp.int32)
    pad_idx = n + (jnp.arange(epad - e, dtype=jnp.int32) % (npad - n))
    src3 = jnp.concatenate([src, pad_idx]).reshape(NW, cp, CH)
    dst3 = jnp.concatenate([dst, pad_idx]).reshape(NW, cp, CH)
    feat_pad = jnp.zeros((npad, f), jnp.float32).at[:n].set(features)

    dp = _deg_call(src3, dst3, npad)
    h1, norms = _tc1(feat_pad, W1, dp)
    z1 = jnp.zeros_like(h1)
    p1 = _agg_call(h1, src3, dst3, z1)
    h2 = _tc2(p1, norms, b1.reshape(1, -1), W2)
    z2 = jnp.zeros_like(h2)
    p2 = _agg_call(h2, src3, dst3, z2)
    return _tc3(p2, norms, b2.reshape(1, -1), n)


# speedup vs baseline: 13.9804x; 1.1012x over previous
import functools

import jax
import jax.numpy as jnp
from jax import lax
from jax.experimental import pallas as pl
from jax.experimental.pallas import tpu as pltpu
from jax.experimental.pallas import tpu_sc as plsc

NC = 2
NS = 16
NW = NC * NS
CH = 128


def _sc_mesh():
    return plsc.VectorSubcoreMesh(core_axis_name="c", subcore_axis_name="s")


def _deg_call(src3, dst3, npad):
    cp = src3.shape[1]
    sl = npad // NS

    @functools.partial(
        pl.kernel,
        mesh=_sc_mesh(),
        out_type=jax.ShapeDtypeStruct((NC, 2, npad), jnp.float32),
        scratch_types=[
            pltpu.VMEM_SHARED((npad,), jnp.float32),
            pltpu.VMEM_SHARED((npad,), jnp.float32),
            pltpu.VMEM((cp, CH), jnp.int32),
            pltpu.VMEM((cp, CH), jnp.int32),
            pltpu.VMEM((CH,), jnp.float32),
            pltpu.VMEM((sl,), jnp.float32),
            pltpu.SemaphoreType.DMA,
        ],
    )
    def k(src_h, dst_h, out_h, dego, degi, sidx, didx, ones_v, zbuf, sem):
        c = lax.axis_index("c")
        s = lax.axis_index("s")
        wid = c * NS + s
        for i in range(sl // 16):
            zbuf[pl.ds(i * 16, 16)] = jnp.zeros((16,), jnp.float32)
        for i in range(CH // 16):
            ones_v[pl.ds(i * 16, 16)] = jnp.ones((16,), jnp.float32)
        pltpu.sync_copy(zbuf, dego.at[pl.ds(s * sl, sl)])
        pltpu.sync_copy(zbuf, degi.at[pl.ds(s * sl, sl)])
        pltpu.sync_copy(src_h.at[wid], sidx)
        pltpu.sync_copy(dst_h.at[wid], didx)
        plsc.subcore_barrier()
        ds_ = []
        for j in range(cp):
            ds_.append(pltpu.async_copy(ones_v, dego.at[sidx.at[j]], sem, add=True))
            ds_.append(pltpu.async_copy(ones_v, degi.at[didx.at[j]], sem, add=True))
        for d_ in ds_:
            d_.wait()
        plsc.subcore_barrier()
        pltpu.sync_copy(dego.at[pl.ds(s * sl, sl)], out_h.at[c, 0, pl.ds(s * sl, sl)])
        pltpu.sync_copy(degi.at[pl.ds(s * sl, sl)], out_h.at[c, 1, pl.ds(s * sl, sl)])

    return k(src3, dst3)


def _agg_call(h, src3, dst3, zeros):
    npad, d = h.shape
    dt = h.dtype
    cp = src3.shape[1]
    sl = npad // NS

    @functools.partial(
        pl.kernel,
        mesh=_sc_mesh(),
        compiler_params=pltpu.CompilerParams(use_tc_tiling_on_sc=False),
        out_type=jax.ShapeDtypeStruct((NC, npad, d), dt),
        scratch_types=[
            pltpu.VMEM_SHARED((npad, d), dt),
            pltpu.VMEM((4, CH), jnp.int32),
            pltpu.VMEM((4, CH), jnp.int32),
            pltpu.VMEM((2, CH, d), dt),
            pltpu.SemaphoreType.DMA,
            [pltpu.SemaphoreType.DMA] * 4,
            [pltpu.SemaphoreType.DMA] * 2,
            [pltpu.SemaphoreType.DMA] * 2,
        ],
    )
    def k(h_h, src_h, dst_h, z_h, out_h, acc, sidx, didx, rows, zsem, isems, gsems, ssems):
        c = lax.axis_index("c")
        s = lax.axis_index("s")
        wid = c * NS + s
        zd = pltpu.async_copy(z_h.at[pl.ds(s * sl, sl)], acc.at[pl.ds(s * sl, sl)], zsem)

        idd, gd, sd = {}, {}, {}

        def load_idx(j):
            r = j % 4
            idd[j] = (
                pltpu.async_copy(src_h.at[wid, pl.ds(j, 1)], sidx.at[pl.ds(r, 1)], isems[r]),
                pltpu.async_copy(dst_h.at[wid, pl.ds(j, 1)], didx.at[pl.ds(r, 1)], isems[r]),
            )

        def gather(j):
            gd[j] = pltpu.async_copy(h_h.at[sidx.at[j % 4]], rows.at[j % 2], gsems[j % 2])

        for t in range(min(3, cp)):
            load_idx(t)
        idd[0][0].wait()
        idd[0][1].wait()
        gather(0)
        zd.wait()
        plsc.subcore_barrier()
        for j in range(cp):
            if j >= 1:
                sd[j - 1].wait()
            if j + 3 < cp:
                load_idx(j + 3)
            if j + 1 < cp:
                idd[j + 1][0].wait()
                idd[j + 1][1].wait()
                gather(j + 1)
            gd[j].wait()
            sd[j] = pltpu.async_copy(
                rows.at[j % 2], acc.at[didx.at[j % 4]], ssems[j % 2], add=True)
        sd[cp - 1].wait()
        plsc.subcore_barrier()
        pltpu.sync_copy(acc.at[pl.ds(s * sl, sl)], out_h.at[c, pl.ds(s * sl, sl)])

    return k(h, src3, dst3, zeros)


def _row_scale(y, ns):
    r, c = y.shape
    eye = (jax.lax.broadcasted_iota(jnp.int32, (128, 128), 0)
           == jax.lax.broadcasted_iota(jnp.int32, (128, 128), 1)).astype(jnp.float32)
    parts = []
    for k in range(r // 128):
        dg = eye * ns[k * 128:(k + 1) * 128][None, :]
        parts.append(jnp.dot(dg, y[k * 128:(k + 1) * 128, :],
                             preferred_element_type=jnp.float32))
    return jnp.concatenate(parts, axis=0)


def _tc1(feat, w1, dp):
    npad, f = feat.shape
    h = w1.shape[1]
    br = 512
    g = npad // br

    def body(f_ref, w_ref, dp_ref, h_ref, n_ref):
        dsrc = dp_ref[0, 0] + dp_ref[1, 0]
        ddst = dp_ref[0, 1] + dp_ref[1, 1]
        ns = lax.rsqrt(jnp.maximum(dsrc, 1.0))
        nd = lax.rsqrt(jnp.maximum(ddst, 1.0))
        n_ref[0, :] = ns
        n_ref[1, :] = nd
        y = jnp.dot(f_ref[...], w_ref[...], preferred_element_type=jnp.float32)
        h_ref[...] = _row_scale(y, ns).astype(jnp.bfloat16)

    return pl.pallas_call(
        body,
        grid=(g,),
        in_specs=[
            pl.BlockSpec((br, f), lambda i: (i, 0)),
            pl.BlockSpec((f, h), lambda i: (0, 0)),
            pl.BlockSpec((NC, 2, br), lambda i: (0, 0, i)),
        ],
        out_specs=[
            pl.BlockSpec((br, h), lambda i: (i, 0)),
            pl.BlockSpec((2, br), lambda i: (0, i)),
        ],
        out_shape=[
            jax.ShapeDtypeStruct((npad, h), jnp.bfloat16),
            jax.ShapeDtypeStruct((2, npad), jnp.float32),
        ],
    )(feat, w1, dp)


def _tc2(p1, norms, b1, w2):
    npad, h = p1.shape[1], p1.shape[2]
    o = w2.shape[1]
    br = 512
    g = npad // br

    def body(p_ref, n_ref, b_ref, w_ref, o_ref):
        agg = p_ref[0].astype(jnp.float32) + p_ref[1].astype(jnp.float32)
        x = jnp.maximum(_row_scale(agg, n_ref[1, :]) + b_ref[...], 0.0)
        y = jnp.dot(x, w_ref[...], preferred_element_type=jnp.float32)
        o_ref[...] = _row_scale(y, n_ref[0, :]).astype(jnp.bfloat16)

    return pl.pallas_call(
        body,
        grid=(g,),
        in_specs=[
            pl.BlockSpec((NC, br, h), lambda i: (0, i, 0)),
            pl.BlockSpec((2, br), lambda i: (0, i)),
            pl.BlockSpec((1, h), lambda i: (0, 0)),
            pl.BlockSpec((h, o), lambda i: (0, 0)),
        ],
        out_specs=pl.BlockSpec((br, o), lambda i: (i, 0)),
        out_shape=jax.ShapeDtypeStruct((npad, o), jnp.bfloat16),
    )(p1, norms, b1, w2)


def _tc3(p2, norms, b2, n_out):
    npad, o = p2.shape[1], p2.shape[2]
    br = 512
    g = npad // br

    def body(p_ref, n_ref, b_ref, o_ref):
        agg = p_ref[0].astype(jnp.float32) + p_ref[1].astype(jnp.float32)
        o_ref[...] = _row_scale(agg, n_ref[1, :]) + b_ref[...]

    return pl.pallas_call(
        body,
        grid=(g,),
        in_specs=[
            pl.BlockSpec((NC, br, o), lambda i: (0, i, 0)),
            pl.BlockSpec((2, br), lambda i: (0, i)),
            pl.BlockSpec((1, o), lambda i: (0, 0)),
        ],
        out_specs=pl.BlockSpec((br, o), lambda i: (i, 0)),
        out_shape=jax.ShapeDtypeStruct((n_out, o), jnp.float32),
    )(p2, norms, b2)


def kernel(features, edge_index, W1, b1, W2, b2):
    n, f = features.shape
    e = edge_index.shape[1]
    npad = -(-(n + 1) // 2560) * 2560
    cp = -(-e // (NW * CH))
    epad = NW * CH * cp

    src = edge_index[0].astype(jnp.int32)
    dst = edge_index[1].astype(jnp.int32)
    pad_idx = n + (jnp.arange(epad - e, dtype=jnp.int32) % (npad - n))
    src3 = jnp.concatenate([src, pad_idx]).reshape(NW, cp, CH)
    dst3 = jnp.concatenate([dst, pad_idx]).reshape(NW, cp, CH)
    feat_pad = jnp.zeros((npad, f), jnp.float32).at[:n].set(features)

    dp = _deg_call(src3, dst3, npad)
    h1, norms = _tc1(feat_pad, W1, dp)
    z1 = jnp.zeros_like(h1)
    p1 = _agg_call(h1, src3, dst3, z1)
    h2 = _tc2(p1, norms, b1.reshape(1, -1), W2)
    z2 = jnp.zeros_like(h2)
    p2 = _agg_call(h2, src3, dst3, z2)
    return _tc3(p2, norms, b2.reshape(1, -1), n)
